# baseline (speedup 1.0000x reference)
import functools

import jax
import jax.numpy as jnp
from jax import lax
from jax.experimental import pallas as pl
from jax.experimental.pallas import tpu as pltpu
from jax.experimental.pallas import tpu_sc as plsc

_NUM_EMBEDDINGS = 100000
_DIM = 64
_BATCH = 1024
_LANES = 128
_NB = 12
_TC_N = 256
_SC_N = _BATCH - _TC_N
_TNB = 8


def _make_sc_gather():
    info = plsc.get_sparse_core_info()
    nc, ns = info.num_cores, info.num_subcores
    nw = nc * ns
    b_per_w = _SC_N // nw
    nl = 16
    mesh = plsc.VectorSubcoreMesh(core_axis_name="c", subcore_axis_name="s")

    @functools.partial(
        pl.kernel,
        mesh=mesh,
        out_type=jax.ShapeDtypeStruct((_SC_N, _DIM), jnp.float32),
        scratch_types=[
            pltpu.VMEM((32,), jnp.int32),
            pltpu.VMEM((_NB, _DIM, _LANES), jnp.float32),
            pltpu.VMEM((b_per_w, _DIM), jnp.float32),
        ] + [pltpu.SemaphoreType.DMA] * _NB,
        compiler_params=pltpu.CompilerParams(needs_layout_passes=False),
    )
    def gather_kernel(idx_hbm, tablet_hbm, out_hbm,
                      idx_v, blk_v, rows_v, *sems):
        wid = lax.axis_index("s") * nc + lax.axis_index("c")
        base = wid * b_per_w
        pltpu.sync_copy(idx_hbm.at[pl.ds(base, 32)], idx_v)
        lanes16 = lax.iota(jnp.int32, nl)
        handles = [None] * b_per_w

        def scalar_id(j):
            v = idx_v[pl.ds((j // nl) * nl, nl)]
            return v[j % nl]

        def fire(j):
            q = lax.shift_right_logical(scalar_id(j), 7)
            off = pl.multiple_of(q * _LANES, _LANES)
            handles[j] = pltpu.async_copy(
                tablet_hbm.at[:, pl.ds(off, _LANES)],
                blk_v.at[j % _NB], sems[j % _NB])

        for j in range(min(_NB, b_per_w)):
            fire(j)
        for j in range(b_per_w):
            handles[j].wait()
            r = jnp.full((nl,), scalar_id(j) & (_LANES - 1), jnp.int32)
            blk_j = blk_v.at[j % _NB]
            for k in range(_DIM // nl):
                dd = lanes16 + (k * nl)
                val = plsc.load_gather(blk_j, [dd, r])
                rows_v[j, pl.ds(k * nl, nl)] = val
            if j + _NB < b_per_w:
                fire(j + _NB)
        pltpu.sync_copy(rows_v, out_hbm.at[pl.ds(base, b_per_w)])

    return gather_kernel


def _tc_select_kernel(ids_smem, tablet_any, out_any, blk_v, rows_v, out_sem,
                      *sems):
    def fire(t):
        q = lax.shift_right_logical(ids_smem[t], 7)
        off = pl.multiple_of(q * _LANES, _LANES)
        return pltpu.make_async_copy(
            tablet_any.at[:, pl.ds(off, _LANES)],
            blk_v.at[t % _TNB], sems[t % _TNB])

    handles = [None] * _TC_N
    for t in range(_TNB):
        handles[t] = fire(t)
        handles[t].start()
    iota_l = lax.broadcasted_iota(jnp.int32, (1, _LANES), 1)
    for t in range(_TC_N):
        handles[t].wait()
        r = ids_smem[t] & (_LANES - 1)
        onehot = (iota_l == r).astype(jnp.float32)
        blk = blk_v[t % _TNB]
        val = lax.dot_general(onehot, blk, (((1,), (1,)), ((), ())),
                              preferred_element_type=jnp.float32)
        rows_v[pl.ds(t, 1), :] = val
        if t + _TNB < _TC_N:
            handles[t + _TNB] = fire(t + _TNB)
            handles[t + _TNB].start()
    done = pltpu.make_async_copy(rows_v, out_any, out_sem)
    done.start()
    done.wait()


def _tc_gather(ids_tc, tablet):
    return pl.pallas_call(
        _tc_select_kernel,
        out_shape=jax.ShapeDtypeStruct((_TC_N, _DIM), jnp.float32),
        in_specs=[
            pl.BlockSpec(memory_space=pltpu.SMEM),
            pl.BlockSpec(memory_space=pltpu.HBM),
        ],
        out_specs=pl.BlockSpec(memory_space=pltpu.HBM),
        scratch_shapes=[
            pltpu.VMEM((_TNB, _DIM, _LANES), jnp.float32),
            pltpu.VMEM((_TC_N, _DIM), jnp.float32),
            pltpu.SemaphoreType.DMA,
        ] + [pltpu.SemaphoreType.DMA] * _TNB,
    )(ids_tc, tablet)


_sc_gather = _make_sc_gather()


def kernel(token_ids, embedding):
    ids = token_ids.astype(jnp.int32)
    tablet = embedding.T
    sc_out = _sc_gather(ids[:_SC_N], tablet)
    tc_out = _tc_gather(ids[_SC_N:], tablet)
    return jnp.concatenate([sc_out, tc_out], axis=0)


# speedup vs baseline: 1.5800x; 1.5800x over previous
import functools

import jax
import jax.numpy as jnp
from jax import lax
from jax.experimental import pallas as pl
from jax.experimental.pallas import tpu as pltpu
from jax.experimental.pallas import tpu_sc as plsc

_NUM_EMBEDDINGS = 100000
_DIM = 64
_BATCH = 1024
_LANES = 128
_NB = 12


def _make_gather():
    info = plsc.get_sparse_core_info()
    nc, ns = info.num_cores, info.num_subcores
    nw = nc * ns
    b_per_w = _BATCH // nw
    nl = 16
    mesh = plsc.VectorSubcoreMesh(core_axis_name="c", subcore_axis_name="s")

    @functools.partial(
        pl.kernel,
        mesh=mesh,
        out_type=jax.ShapeDtypeStruct((_BATCH, _DIM), jnp.float32),
        scratch_types=[
            pltpu.VMEM((b_per_w,), jnp.int32),
            pltpu.VMEM((_NB, _DIM, _LANES), jnp.float32),
            pltpu.VMEM((b_per_w, _DIM), jnp.float32),
        ] + [pltpu.SemaphoreType.DMA] * _NB,
        compiler_params=pltpu.CompilerParams(needs_layout_passes=False),
    )
    def gather_kernel(idx_hbm, tablet_hbm, out_hbm,
                      idx_v, blk_v, rows_v, *sems):
        wid = lax.axis_index("s") * nc + lax.axis_index("c")
        base = wid * b_per_w
        pltpu.sync_copy(idx_hbm.at[pl.ds(base, b_per_w)], idx_v)
        lanes16 = lax.iota(jnp.int32, nl)
        handles = [None] * b_per_w

        def scalar_id(j):
            v = idx_v[pl.ds((j // nl) * nl, nl)]
            return v[j % nl]

        def fire(j):
            q = lax.shift_right_logical(scalar_id(j), 7)
            off = pl.multiple_of(q * _LANES, _LANES)
            handles[j] = pltpu.async_copy(
                tablet_hbm.at[:, pl.ds(off, _LANES)],
                blk_v.at[j % _NB], sems[j % _NB])

        for j in range(_NB):
            fire(j)
        for j in range(b_per_w):
            handles[j].wait()
            r = jnp.full((nl,), scalar_id(j) & (_LANES - 1), jnp.int32)
            blk_j = blk_v.at[j % _NB]
            for k in range(_DIM // nl):
                dd = lanes16 + (k * nl)
                val = plsc.load_gather(blk_j, [dd, r])
                rows_v[j, pl.ds(k * nl, nl)] = val
            if j + _NB < b_per_w:
                fire(j + _NB)
        pltpu.sync_copy(rows_v, out_hbm.at[pl.ds(base, b_per_w)])

    return gather_kernel


_gather = _make_gather()


def kernel(token_ids, embedding):
    return _gather(token_ids.astype(jnp.int32), embedding.T)
